# baseline (speedup 1.0000x reference)
import functools

import jax
import jax.numpy as jnp
from jax import lax
from jax.experimental import pallas as pl
from jax.experimental.pallas import tpu as pltpu
from jax.experimental.pallas import tpu_sc as plsc

VOCAB = 30522
EMB = 128
HID = 480
B, S = 128, 512
TOKENS = B * S
EPS = 1e-9

_NC, _NS = 2, 16
_NW = _NC * _NS
_ROWS_PER_XFER = 128
_XFERS_PER_W = TOKENS // (_NW * _ROWS_PER_XFER)


def _sc_gather_body(idx_hbm, table_hbm, out_hbm, idx_v, rows_v, sem0, sem1):
    wid = lax.axis_index("s") * _NC + lax.axis_index("c")
    row0 = wid * _XFERS_PER_W
    tok0 = row0 * _ROWS_PER_XFER
    pltpu.sync_copy(idx_hbm.at[pl.ds(row0, _XFERS_PER_W)], idx_v)
    sems = (sem0, sem1)
    copies = [None, None]
    for j in range(_XFERS_PER_W):
        copies[j % 2] = pltpu.async_copy(
            table_hbm.at[idx_v.at[j]], rows_v.at[j % 2], sems[j % 2])
        if j > 0:
            copies[(j - 1) % 2].wait()
            pltpu.sync_copy(
                rows_v.at[(j - 1) % 2],
                out_hbm.at[pl.ds(tok0 + (j - 1) * _ROWS_PER_XFER,
                                 _ROWS_PER_XFER)])
    j = _XFERS_PER_W - 1
    copies[j % 2].wait()
    pltpu.sync_copy(
        rows_v.at[j % 2],
        out_hbm.at[pl.ds(tok0 + j * _ROWS_PER_XFER, _ROWS_PER_XFER)])


@functools.cache
def _sc_gather():
    return pl.kernel(
        _sc_gather_body,
        out_type=jax.ShapeDtypeStruct((TOKENS, EMB), jnp.float32),
        mesh=plsc.VectorSubcoreMesh(
            core_axis_name="c", subcore_axis_name="s",
            num_cores=_NC, num_subcores=_NS),
        scratch_types=[
            pltpu.VMEM((_XFERS_PER_W, _ROWS_PER_XFER), jnp.int32),
            pltpu.VMEM((2, _ROWS_PER_XFER, EMB), jnp.float32),
            pltpu.SemaphoreType.DMA,
            pltpu.SemaphoreType.DMA,
        ],
    )

_TB = 2048


def _tc_body(emb_ref, pos_ref, w_ref, b_ref, g_ref, beta_ref, out_ref):
    x = emb_ref[...]
    x = (x.reshape(_TB // S, S, EMB) + pos_ref[...][None]).reshape(_TB, EMB)
    h = jnp.dot(x, w_ref[...], preferred_element_type=jnp.float32)
    h = h + b_ref[...]
    mu = jnp.mean(h, axis=-1, keepdims=True)
    d = h - mu
    var = jnp.mean(d * d, axis=-1, keepdims=True)
    out_ref[...] = (d * lax.rsqrt(var + EPS) * g_ref[...]
                    + beta_ref[...]).reshape(_TB // S, S, HID)


def _tc_fused(emb, pos, w, b2, g2, beta2):
    return pl.pallas_call(
        _tc_body,
        grid=(TOKENS // _TB,),
        in_specs=[
            pl.BlockSpec((_TB, EMB), lambda i: (i, 0)),
            pl.BlockSpec((S, EMB), lambda i: (0, 0)),
            pl.BlockSpec((EMB, HID), lambda i: (0, 0)),
            pl.BlockSpec((1, HID), lambda i: (0, 0)),
            pl.BlockSpec((1, HID), lambda i: (0, 0)),
            pl.BlockSpec((1, HID), lambda i: (0, 0)),
        ],
        out_specs=pl.BlockSpec((_TB // S, S, HID), lambda i: (i, 0, 0)),
        out_shape=jax.ShapeDtypeStruct((B, S, HID), jnp.float32),
    )(emb, pos, w, b2, g2, beta2)


@jax.jit
def kernel(input_ids, word_emb, pos_emb, proj_w, proj_b, ln_g, ln_b):
    idx = input_ids.reshape(TOKENS // _ROWS_PER_XFER, _ROWS_PER_XFER)
    gathered = _sc_gather()(idx, word_emb)
    pos = pos_emb[:S]
    return _tc_fused(gathered, pos, proj_w,
                     proj_b.reshape(1, HID), ln_g.reshape(1, HID),
                     ln_b.reshape(1, HID))


# speedup vs baseline: 1.2879x; 1.2879x over previous
import functools

import jax
import jax.numpy as jnp
from jax import lax
from jax.experimental import pallas as pl
from jax.experimental.pallas import tpu as pltpu
from jax.experimental.pallas import tpu_sc as plsc

VOCAB = 30522
EMB = 128
HID = 480
B, S = 128, 512
TOKENS = B * S
EPS = 1e-9

_NC, _NS = 2, 16
_NW = _NC * _NS
_ROWS_PER_XFER = 128
_XFERS_PER_W = TOKENS // (_NW * _ROWS_PER_XFER)


def _sc_gather_body(idx_hbm, table_hbm, out_hbm, idx_v, rows_v, sem0, sem1):
    wid = lax.axis_index("s") * _NC + lax.axis_index("c")
    row0 = wid * _XFERS_PER_W
    tok0 = row0 * _ROWS_PER_XFER
    pltpu.sync_copy(idx_hbm.at[pl.ds(row0, _XFERS_PER_W)], idx_v)
    sems = (sem0, sem1)
    copies = [None, None]
    for j in range(_XFERS_PER_W):
        copies[j % 2] = pltpu.async_copy(
            table_hbm.at[idx_v.at[j]], rows_v.at[j % 2], sems[j % 2])
        if j > 0:
            copies[(j - 1) % 2].wait()
            pltpu.sync_copy(
                rows_v.at[(j - 1) % 2],
                out_hbm.at[pl.ds(tok0 + (j - 1) * _ROWS_PER_XFER,
                                 _ROWS_PER_XFER)])
    j = _XFERS_PER_W - 1
    copies[j % 2].wait()
    pltpu.sync_copy(
        rows_v.at[j % 2],
        out_hbm.at[pl.ds(tok0 + j * _ROWS_PER_XFER, _ROWS_PER_XFER)])


@functools.cache
def _sc_gather():
    return pl.kernel(
        _sc_gather_body,
        out_type=jax.ShapeDtypeStruct((TOKENS, EMB), jnp.float32),
        mesh=plsc.VectorSubcoreMesh(
            core_axis_name="c", subcore_axis_name="s",
            num_cores=_NC, num_subcores=_NS),
        scratch_types=[
            pltpu.VMEM((_XFERS_PER_W, _ROWS_PER_XFER), jnp.int32),
            pltpu.VMEM((2, _ROWS_PER_XFER, EMB), jnp.float32),
            pltpu.SemaphoreType.DMA,
            pltpu.SemaphoreType.DMA,
        ],
    )

def _tc_body(emb_ref, pos_ref, wT_ref, b_ref, g_ref, beta_ref, out_ref):
    x = emb_ref[...] + pos_ref[...]
    h = lax.dot_general(wT_ref[...], x, (((1,), (1,)), ((), ())),
                        preferred_element_type=jnp.float32)
    h = h + b_ref[...]
    mu = jnp.mean(h, axis=0, keepdims=True)
    d = h - mu
    var = jnp.mean(d * d, axis=0, keepdims=True)
    out_ref[...] = (d * lax.rsqrt(var + EPS) * g_ref[...]
                    + beta_ref[...]).reshape(1, HID, S)


def _tc_fused(emb, pos, wT, b2, g2, beta2):
    return pl.pallas_call(
        _tc_body,
        grid=(B,),
        in_specs=[
            pl.BlockSpec((S, EMB), lambda i: (i, 0)),
            pl.BlockSpec((S, EMB), lambda i: (0, 0)),
            pl.BlockSpec((HID, EMB), lambda i: (0, 0)),
            pl.BlockSpec((HID, 1), lambda i: (0, 0)),
            pl.BlockSpec((HID, 1), lambda i: (0, 0)),
            pl.BlockSpec((HID, 1), lambda i: (0, 0)),
        ],
        out_specs=pl.BlockSpec((1, HID, S), lambda i: (i, 0, 0)),
        out_shape=jax.ShapeDtypeStruct((B, HID, S), jnp.float32),
    )(emb, pos, wT, b2, g2, beta2)


@jax.jit
def kernel(input_ids, word_emb, pos_emb, proj_w, proj_b, ln_g, ln_b):
    idx = input_ids.reshape(TOKENS // _ROWS_PER_XFER, _ROWS_PER_XFER)
    gathered = _sc_gather()(idx, word_emb)
    pos = pos_emb[:S]
    out_t = _tc_fused(gathered, pos, proj_w.T,
                      proj_b.reshape(HID, 1), ln_g.reshape(HID, 1),
                      ln_b.reshape(HID, 1))
    return jnp.transpose(out_t, (0, 2, 1))


# speedup vs baseline: 1.8492x; 1.4358x over previous
import functools

import jax
import jax.numpy as jnp
from jax import lax
from jax.experimental import pallas as pl
from jax.experimental.pallas import tpu as pltpu
from jax.experimental.pallas import tpu_sc as plsc

VOCAB = 30522
EMB = 128
HID = 480
B, S = 128, 512
TOKENS = B * S
EPS = 1e-9

_NC, _NS = 2, 16
_NW = _NC * _NS
_ROWS_PER_XFER = 128
_XFERS_PER_W = TOKENS // (_NW * _ROWS_PER_XFER)


def _sc_gather_body(idx_hbm, table_hbm, out_hbm, idx_v, rows_v, sem0, sem1):
    wid = lax.axis_index("s") * _NC + lax.axis_index("c")
    row0 = wid * _XFERS_PER_W
    tok0 = row0 * _ROWS_PER_XFER
    pltpu.sync_copy(idx_hbm.at[pl.ds(row0, _XFERS_PER_W)], idx_v)
    sems = (sem0, sem1)
    copies = [None, None]
    for j in range(_XFERS_PER_W):
        copies[j % 2] = pltpu.async_copy(
            table_hbm.at[idx_v.at[j]], rows_v.at[j % 2], sems[j % 2])
        if j > 0:
            copies[(j - 1) % 2].wait()
            pltpu.sync_copy(
                rows_v.at[(j - 1) % 2],
                out_hbm.at[pl.ds(tok0 + (j - 1) * _ROWS_PER_XFER,
                                 _ROWS_PER_XFER)])
    j = _XFERS_PER_W - 1
    copies[j % 2].wait()
    pltpu.sync_copy(
        rows_v.at[j % 2],
        out_hbm.at[pl.ds(tok0 + j * _ROWS_PER_XFER, _ROWS_PER_XFER)])


@functools.cache
def _sc_gather():
    return pl.kernel(
        _sc_gather_body,
        out_type=jax.ShapeDtypeStruct((TOKENS, EMB), jnp.float32),
        mesh=plsc.VectorSubcoreMesh(
            core_axis_name="c", subcore_axis_name="s",
            num_cores=_NC, num_subcores=_NS),
        scratch_types=[
            pltpu.VMEM((_XFERS_PER_W, _ROWS_PER_XFER), jnp.int32),
            pltpu.VMEM((2, _ROWS_PER_XFER, EMB), jnp.float32),
            pltpu.SemaphoreType.DMA,
            pltpu.SemaphoreType.DMA,
        ],
    )

_SEQ_PER_STEP = 4


def _tc_body(emb_ref, pos_ref, wT_ref, b_ref, g_ref, beta_ref, out_ref):
    wT = wT_ref[...]
    for k in range(_SEQ_PER_STEP):
        x = emb_ref[pl.ds(k * S, S), :] + pos_ref[...]
        h = lax.dot_general(wT, x, (((1,), (1,)), ((), ())),
                            preferred_element_type=jnp.float32)
        h = h + b_ref[...]
        mu = jnp.mean(h, axis=0, keepdims=True)
        d = h - mu
        var = jnp.mean(d * d, axis=0, keepdims=True)
        out_ref[k, :, :] = (d * lax.rsqrt(var + EPS) * g_ref[...]
                            + beta_ref[...])


def _tc_fused(emb, pos, wT, b2, g2, beta2):
    return pl.pallas_call(
        _tc_body,
        grid=(B // _SEQ_PER_STEP,),
        in_specs=[
            pl.BlockSpec((_SEQ_PER_STEP * S, EMB), lambda i: (i, 0)),
            pl.BlockSpec((S, EMB), lambda i: (0, 0)),
            pl.BlockSpec((HID, EMB), lambda i: (0, 0)),
            pl.BlockSpec((HID, 1), lambda i: (0, 0)),
            pl.BlockSpec((HID, 1), lambda i: (0, 0)),
            pl.BlockSpec((HID, 1), lambda i: (0, 0)),
        ],
        out_specs=pl.BlockSpec((_SEQ_PER_STEP, HID, S), lambda i: (i, 0, 0)),
        out_shape=jax.ShapeDtypeStruct((B, HID, S), jnp.float32),
    )(emb, pos, wT, b2, g2, beta2)


@jax.jit
def kernel(input_ids, word_emb, pos_emb, proj_w, proj_b, ln_g, ln_b):
    idx = input_ids.reshape(TOKENS // _ROWS_PER_XFER, _ROWS_PER_XFER)
    gathered = _sc_gather()(idx, word_emb)
    pos = pos_emb[:S]
    out_t = _tc_fused(gathered, pos, proj_w.T,
                      proj_b.reshape(HID, 1), ln_g.reshape(HID, 1),
                      ln_b.reshape(HID, 1))
    return jnp.transpose(out_t, (0, 2, 1))
